# baseline (speedup 1.0000x reference)
import functools

import jax
import jax.numpy as jnp
from jax import lax
from jax.experimental import pallas as pl
from jax.experimental.pallas import tpu as pltpu
from jax.experimental.pallas import tpu_sc as plsc


def _logits_body(x_ref, rw_ref, out_ref):
    out_ref[...] = lax.dot_general(
        x_ref[...], rw_ref[...],
        (((1,), (1,)), ((), ())),
        preferred_element_type=jnp.float32,
    )


def _router_logits(x, rw):
    n, h = x.shape
    e = rw.shape[0]
    return pl.pallas_call(
        _logits_body,
        out_shape=jax.ShapeDtypeStruct((n, e), jnp.float32),
    )(x, rw)


def _make_sc_router(n, e, k):
    info = plsc.get_sparse_core_info()
    nc, ns, lanes = info.num_cores, info.num_subcores, info.num_lanes
    assert e == lanes, "one token's logits must fill one SC vreg"
    nw = nc * ns
    assert n % nw == 0
    tpw = n // nw

    mesh = plsc.VectorSubcoreMesh(core_axis_name="c", subcore_axis_name="s")

    @functools.partial(
        pl.kernel,
        mesh=mesh,
        out_type=jax.ShapeDtypeStruct((n, e), jnp.float32),
        scratch_types=[
            pltpu.VMEM((tpw, e), jnp.float32),
            pltpu.VMEM((tpw, e), jnp.float32),
        ],
        compiler_params=pltpu.CompilerParams(needs_layout_passes=False),
    )
    def sc_router(logits_hbm, cw_hbm, lg_v, cw_v):
        wid = lax.axis_index("s") * nc + lax.axis_index("c")
        base = wid * tpw
        pltpu.sync_copy(logits_hbm.at[pl.ds(base, tpw)], lg_v)
        lane_ids = lax.iota(jnp.int32, lanes)
        top_mask = lane_ids >= (lanes - k)

        @plsc.parallel_loop(0, tpw, 1, unroll=4)
        def body(t):
            lg = lg_v[t]
            ex = jnp.exp(lg)
            z = jnp.sum(ex, axis=0)
            sk, sv = plsc.sort_key_val(ex, lane_ids)
            s_sel = jnp.sum(jnp.where(top_mask, sk, 0.0), axis=0)
            w = jnp.where(top_mask, sk / (s_sel + 1e-6 * z), 0.0)
            _, row = plsc.sort_key_val(sv, w)
            cw_v[t] = row
        pltpu.sync_copy(cw_v, cw_hbm.at[pl.ds(base, tpw)])

    return sc_router


def _moe_body(x_ref, w_ref, cw_ref, out_ref):
    gi = pl.program_id(0)
    e = cw_ref.shape[1]
    xb = x_ref[...].astype(jnp.bfloat16)
    lanes_e = lax.broadcasted_iota(jnp.int32, (1, e), 1)
    contrib = None
    for j in range(w_ref.shape[0]):
        acc = lax.dot_general(
            xb, w_ref[j].astype(jnp.bfloat16),
            (((1,), (0,)), ((), ())),
            preferred_element_type=jnp.float32,
        )
        ei = gi * w_ref.shape[0] + j
        onehot = (lanes_e == ei).astype(jnp.float32)
        col = jnp.sum(cw_ref[...] * onehot, axis=1, keepdims=True)
        part = acc * col
        contrib = part if contrib is None else contrib + part

    @pl.when(gi == 0)
    def _():
        out_ref[...] = contrib

    @pl.when(gi > 0)
    def _():
        out_ref[...] += contrib


def _moe_combine(x, w, cw, experts_per_step=4):
    n, h = x.shape
    e = w.shape[0]
    eps = experts_per_step
    return pl.pallas_call(
        _moe_body,
        grid=(e // eps,),
        in_specs=[
            pl.BlockSpec((n, h), lambda i: (0, 0)),
            pl.BlockSpec((eps, h, h), lambda i: (i, 0, 0)),
            pl.BlockSpec((n, e), lambda i: (0, 0)),
        ],
        out_specs=pl.BlockSpec((n, h), lambda i: (0, 0)),
        out_shape=jax.ShapeDtypeStruct((n, h), jnp.float32),
    )(x, w, cw)


def kernel(hidden_states, cluster_axis, router_w, expert_weights):
    bq, sq, hq = hidden_states.shape
    e = router_w.shape[0]
    k = 8
    x = hidden_states.reshape(-1, hq)
    n = x.shape[0]

    logits = _router_logits(x, router_w)
    cw = _make_sc_router(n, e, k)(logits)
    out = _moe_combine(x, expert_weights, cw)
    return out.reshape(bq, sq, hq)
